# Initial kernel scaffold; baseline (speedup 1.0000x reference)
import jax
import jax.numpy as jnp
from jax.experimental import pallas as pl


def kernel(batch_features, batch_graphs, mlp_W1, mlp_b1, bn_in_gamma, bn_in_beta, mlp_W2, mlp_b2, outer_gamma, outer_beta, pred_W, pred_b, eps):
    raise NotImplementedError("write your pallas kernel here")


# speedup vs baseline: 4.5646x; 4.5646x over previous
import functools

import jax
import jax.numpy as jnp
from jax import lax
from jax.experimental import pallas as pl
from jax.experimental.pallas import tpu as pltpu
from jax.experimental.pallas import tpu_sc as plsc

_N, _E, _D, _H, _OUT, _L = 10000, 320000, 128, 128, 64, 4
_NP = 10240
_NC, _NS = 2, 16
_NT = _NC * _NS
_EPT = _E // _NT
_CH = 80
_NCHUNK = _EPT // _CH
_RPT = _NP // _NS
_ZROWS = 128
_NZ = _RPT // _ZROWS


@functools.cache
def _build_segsum():
    mesh = plsc.VectorSubcoreMesh(core_axis_name="c", subcore_axis_name="s")

    @functools.partial(
        pl.kernel,
        mesh=mesh,
        out_type=jax.ShapeDtypeStruct((_NC, _NP, _H), jnp.float32),
        scratch_types=[
            pltpu.VMEM((_CH,), jnp.int32),
            pltpu.VMEM((_CH,), jnp.int32),
            pltpu.VMEM((_CH, _H), jnp.float32),
            pltpu.VMEM((_ZROWS, _H), jnp.float32),
            pltpu.VMEM_SHARED((_NP, _H), jnp.float32),
            pltpu.SemaphoreType.DMA,
        ],
    )
    def segsum(h_hbm, src_hbm, dst_hbm, out_hbm, src_v, dst_v, rows_v, stg_v,
               acc_sh, sem):
        c = lax.axis_index("c")
        s = lax.axis_index("s")
        tid = s * _NC + c

        zero16 = jnp.zeros((16,), jnp.float32)

        def _zrow(i, carry):
            for j in range(_H // 16):
                stg_v[i, pl.ds(j * 16, 16)] = zero16
            return carry

        lax.fori_loop(0, _ZROWS, _zrow, 0)
        for w in range(_NZ):
            pltpu.sync_copy(stg_v,
                            acc_sh.at[pl.ds(s * _RPT + w * _ZROWS, _ZROWS)])
        plsc.subcore_barrier()

        def _chunk(ci, carry):
            base = pl.multiple_of(tid * _EPT + ci * _CH, 8)
            pltpu.sync_copy(src_hbm.at[pl.ds(base, _CH)], src_v)
            pltpu.sync_copy(dst_hbm.at[pl.ds(base, _CH)], dst_v)
            pltpu.async_copy(h_hbm.at[src_v], rows_v, sem).wait()
            pltpu.sync_copy(rows_v, acc_sh.at[dst_v], add=True)
            return carry

        lax.fori_loop(0, _NCHUNK, _chunk, 0)
        plsc.subcore_barrier()

        for w in range(_NZ):
            r0 = s * _RPT + w * _ZROWS
            pltpu.sync_copy(acc_sh.at[pl.ds(r0, _ZROWS)], stg_v)
            pltpu.sync_copy(stg_v, out_hbm.at[c, pl.ds(r0, _ZROWS)])

    return segsum


def _bn_relu(z, g, b):
    mu = jnp.mean(z, axis=0, keepdims=True)
    var = jnp.mean(z * z, axis=0, keepdims=True) - mu * mu
    inv = lax.rsqrt(var + 1e-5)
    return jnp.maximum(g * (z - mu) * inv + b, 0.0)


def _dense0_body(scale_ref, h_ref, agg_ref, W1_ref, b1_ref, g1_ref, be1_ref,
                 W2_ref, b2_ref, pW0_ref, pb0_ref, pW1_ref, pb1_ref,
                 hout_ref, sout_ref):
    h = h_ref[...]
    a0 = agg_ref[0, pl.ds(0, _N), :]
    a1 = agg_ref[1, pl.ds(0, _N), :]
    pooled = a0 + a1 + scale_ref[...] * h
    z = jnp.dot(pooled, W1_ref[...], preferred_element_type=jnp.float32)
    z = _bn_relu(z + b1_ref[...], g1_ref[...], be1_ref[...])
    z = jnp.dot(z, W2_ref[...], preferred_element_type=jnp.float32)
    z = jnp.maximum(z + b2_ref[...], 0.0)
    hout_ref[...] = z
    s0 = jnp.dot(h, pW0_ref[...], preferred_element_type=jnp.float32)
    s1 = jnp.dot(z, pW1_ref[...], preferred_element_type=jnp.float32)
    sout_ref[...] = s0 + pb0_ref[...] + s1 + pb1_ref[...]


def _denseK_body(scale_ref, h_ref, agg_ref, W1_ref, b1_ref, g1_ref, be1_ref,
                 W2_ref, b2_ref, g2_ref, be2_ref, pW_ref, pb_ref, sin_ref,
                 hout_ref, sout_ref):
    h = h_ref[...]
    a0 = agg_ref[0, pl.ds(0, _N), :]
    a1 = agg_ref[1, pl.ds(0, _N), :]
    pooled = a0 + a1 + scale_ref[...] * h
    z = jnp.dot(pooled, W1_ref[...], preferred_element_type=jnp.float32)
    z = _bn_relu(z + b1_ref[...], g1_ref[...], be1_ref[...])
    z = jnp.dot(z, W2_ref[...], preferred_element_type=jnp.float32)
    z = _bn_relu(z + b2_ref[...], g2_ref[...], be2_ref[...])
    hout_ref[...] = z
    s = jnp.dot(z, pW_ref[...], preferred_element_type=jnp.float32)
    sout_ref[...] = sin_ref[...] + s + pb_ref[...]


_DENSE_OUT = [
    jax.ShapeDtypeStruct((_N, _H), jnp.float32),
    jax.ShapeDtypeStruct((_N, _OUT), jnp.float32),
]


def kernel(batch_features, batch_graphs, mlp_W1, mlp_b1, bn_in_gamma,
           bn_in_beta, mlp_W2, mlp_b2, outer_gamma, outer_beta, pred_W,
           pred_b, eps):
    src = batch_graphs[0]
    dst = batch_graphs[1]
    h = batch_features
    score = None
    for i in range(_L - 1):
        agg = _build_segsum()(h, src, dst)
        scale = (1.0 + eps[i]).reshape(1, 1).astype(jnp.float32)
        if i == 0:
            h, score = pl.pallas_call(_dense0_body, out_shape=_DENSE_OUT)(
                scale, h, agg, mlp_W1[0], mlp_b1[0].reshape(1, _H),
                bn_in_gamma[0].reshape(1, _H), bn_in_beta[0].reshape(1, _H),
                mlp_W2[0], mlp_b2[0].reshape(1, _H),
                pred_W[0], pred_b[0].reshape(1, _OUT),
                pred_W[1], pred_b[1].reshape(1, _OUT))
        else:
            h, score = pl.pallas_call(_denseK_body, out_shape=_DENSE_OUT)(
                scale, h, agg, mlp_W1[i], mlp_b1[i].reshape(1, _H),
                bn_in_gamma[i].reshape(1, _H), bn_in_beta[i].reshape(1, _H),
                mlp_W2[i], mlp_b2[i].reshape(1, _H),
                outer_gamma[i - 1].reshape(1, _H),
                outer_beta[i - 1].reshape(1, _H),
                pred_W[i + 1], pred_b[i + 1].reshape(1, _OUT), score)
    return score


# speedup vs baseline: 11.1927x; 2.4521x over previous
import functools

import jax
import jax.numpy as jnp
from jax import lax
from jax.experimental import pallas as pl
from jax.experimental.pallas import tpu as pltpu
from jax.experimental.pallas import tpu_sc as plsc

_N, _E, _D, _H, _OUT, _L = 10000, 320000, 128, 128, 64, 4
_NP = 10240
_NC, _NS = 2, 16
_NT = _NC * _NS
_EPT = _E // _NT
_CH = 96
_NFULL = _EPT // _CH
_TAIL = _EPT - _NFULL * _CH
_RPT = _NP // _NS
_ZROWS = 80
_NZ = _RPT // _ZROWS


@functools.cache
def _build_segsum():
    mesh = plsc.VectorSubcoreMesh(core_axis_name="c", subcore_axis_name="s")

    @functools.partial(
        pl.kernel,
        mesh=mesh,
        out_type=jax.ShapeDtypeStruct((_NC, _NP, _H), jnp.float32),
        scratch_types=[
            pltpu.VMEM((_EPT,), jnp.int32),
            pltpu.VMEM((_EPT,), jnp.int32),
            pltpu.VMEM((_CH, _H), jnp.float32),
            pltpu.VMEM((_CH, _H), jnp.float32),
            pltpu.VMEM((_CH,), jnp.int32),
            pltpu.VMEM((_CH,), jnp.int32),
            pltpu.VMEM((_TAIL, _H), jnp.float32),
            pltpu.VMEM((_TAIL,), jnp.int32),
            pltpu.VMEM_SHARED((_NP, _H), jnp.float32),
            pltpu.SemaphoreType.DMA,
            pltpu.SemaphoreType.DMA,
        ],
    )
    def segsum(h_hbm, src_hbm, dst_hbm, out_hbm, src_all, dst_all, rows0,
               rows1, dstb0, dstb1, rows_t, dst_t, acc_sh, sem0, sem1):
        c = lax.axis_index("c")
        s = lax.axis_index("s")
        tid = s * _NC + c
        ebase = pl.multiple_of(tid * _EPT, 8)

        pltpu.sync_copy(src_hbm.at[pl.ds(ebase, _EPT)], src_all)
        pltpu.sync_copy(dst_hbm.at[pl.ds(ebase, _EPT)], dst_all)

        zero16 = jnp.zeros((16,), jnp.float32)

        def _zrow(i, carry):
            for j in range(_H // 16):
                rows0[i, pl.ds(j * 16, 16)] = zero16
            return carry

        lax.fori_loop(0, _ZROWS, _zrow, 0)
        zsrc = rows0.at[pl.ds(0, _ZROWS)]
        zcp = []
        for w in range(_NZ):
            zcp.append(pltpu.async_copy(
                zsrc, acc_sh.at[pl.ds(s * _RPT + w * _ZROWS, _ZROWS)], sem0))
        for cp in zcp:
            cp.wait()
        plsc.subcore_barrier()

        def _gather(ci, rows, sem):
            idx = src_all.at[pl.ds(ci * _CH, _CH)]
            return pltpu.async_copy(h_hbm.at[idx], rows, sem)

        def _gwait(ci, rows, sem):
            idx = src_all.at[pl.ds(ci * _CH, _CH)]
            pltpu.make_async_copy(h_hbm.at[idx], rows, sem).wait()

        def _stage(ci, dstb):
            for j in range(_CH // 16):
                dstb[pl.ds(j * 16, 16)] = dst_all[pl.ds(ci * _CH + j * 16, 16)]

        def _consume(ci, rows, dstb, sem):
            _gwait(ci, rows, sem)
            _stage(ci, dstb)
            pltpu.sync_copy(rows, acc_sh.at[dstb], add=True)

        _gather(0, rows0, sem0)
        _gather(1, rows1, sem1)

        def _pair(k, carry):
            c0 = k * 2
            _consume(c0, rows0, dstb0, sem0)
            _gather(c0 + 2, rows0, sem0)
            _consume(c0 + 1, rows1, dstb1, sem1)
            _gather(c0 + 3, rows1, sem1)
            return carry

        lax.fori_loop(0, (_NFULL - 2) // 2, _pair, 0)
        _consume(_NFULL - 2, rows0, dstb0, sem0)
        _consume(_NFULL - 1, rows1, dstb1, sem1)

        tbase = _NFULL * _CH
        pltpu.async_copy(
            h_hbm.at[src_all.at[pl.ds(tbase, _TAIL)]], rows_t, sem0).wait()
        dst_t[...] = dst_all[pl.ds(tbase, _TAIL)]
        pltpu.sync_copy(rows_t, acc_sh.at[dst_t], add=True)
        plsc.subcore_barrier()

        handles = [None, None]
        bufs = [rows0.at[pl.ds(0, _ZROWS)], rows1.at[pl.ds(0, _ZROWS)]]
        sems = [sem0, sem1]
        for w in range(_NZ):
            if handles[w % 2] is not None:
                handles[w % 2].wait()
            r0 = s * _RPT + w * _ZROWS
            pltpu.sync_copy(acc_sh.at[pl.ds(r0, _ZROWS)], bufs[w % 2])
            handles[w % 2] = pltpu.async_copy(
                bufs[w % 2], out_hbm.at[c, pl.ds(r0, _ZROWS)], sems[w % 2])
        handles[0].wait()
        handles[1].wait()

    return segsum


def _bn_relu(z, g, b):
    mu = jnp.mean(z, axis=0, keepdims=True)
    var = jnp.mean(z * z, axis=0, keepdims=True) - mu * mu
    inv = lax.rsqrt(var + 1e-5)
    return jnp.maximum(g * (z - mu) * inv + b, 0.0)


def _dense0_body(scale_ref, h_ref, agg_ref, W1_ref, b1_ref, g1_ref, be1_ref,
                 W2_ref, b2_ref, pW0_ref, pb0_ref, pW1_ref, pb1_ref,
                 hout_ref, sout_ref):
    h = h_ref[...]
    a0 = agg_ref[0, pl.ds(0, _N), :]
    a1 = agg_ref[1, pl.ds(0, _N), :]
    pooled = a0 + a1 + scale_ref[...] * h
    z = jnp.dot(pooled, W1_ref[...], preferred_element_type=jnp.float32)
    z = _bn_relu(z + b1_ref[...], g1_ref[...], be1_ref[...])
    z = jnp.dot(z, W2_ref[...], preferred_element_type=jnp.float32)
    z = jnp.maximum(z + b2_ref[...], 0.0)
    hout_ref[...] = z
    s0 = jnp.dot(h, pW0_ref[...], preferred_element_type=jnp.float32)
    s1 = jnp.dot(z, pW1_ref[...], preferred_element_type=jnp.float32)
    sout_ref[...] = s0 + pb0_ref[...] + s1 + pb1_ref[...]


def _denseK_body(scale_ref, h_ref, agg_ref, W1_ref, b1_ref, g1_ref, be1_ref,
                 W2_ref, b2_ref, g2_ref, be2_ref, pW_ref, pb_ref, sin_ref,
                 hout_ref, sout_ref):
    h = h_ref[...]
    a0 = agg_ref[0, pl.ds(0, _N), :]
    a1 = agg_ref[1, pl.ds(0, _N), :]
    pooled = a0 + a1 + scale_ref[...] * h
    z = jnp.dot(pooled, W1_ref[...], preferred_element_type=jnp.float32)
    z = _bn_relu(z + b1_ref[...], g1_ref[...], be1_ref[...])
    z = jnp.dot(z, W2_ref[...], preferred_element_type=jnp.float32)
    z = _bn_relu(z + b2_ref[...], g2_ref[...], be2_ref[...])
    hout_ref[...] = z
    s = jnp.dot(z, pW_ref[...], preferred_element_type=jnp.float32)
    sout_ref[...] = sin_ref[...] + s + pb_ref[...]


_DENSE_OUT = [
    jax.ShapeDtypeStruct((_N, _H), jnp.float32),
    jax.ShapeDtypeStruct((_N, _OUT), jnp.float32),
]


def kernel(batch_features, batch_graphs, mlp_W1, mlp_b1, bn_in_gamma,
           bn_in_beta, mlp_W2, mlp_b2, outer_gamma, outer_beta, pred_W,
           pred_b, eps):
    src = batch_graphs[0]
    dst = batch_graphs[1]
    h = batch_features
    score = None
    for i in range(_L - 1):
        agg = _build_segsum()(h, src, dst)
        scale = (1.0 + eps[i]).reshape(1, 1).astype(jnp.float32)
        if i == 0:
            h, score = pl.pallas_call(_dense0_body, out_shape=_DENSE_OUT)(
                scale, h, agg, mlp_W1[0], mlp_b1[0].reshape(1, _H),
                bn_in_gamma[0].reshape(1, _H), bn_in_beta[0].reshape(1, _H),
                mlp_W2[0], mlp_b2[0].reshape(1, _H),
                pred_W[0], pred_b[0].reshape(1, _OUT),
                pred_W[1], pred_b[1].reshape(1, _OUT))
        else:
            h, score = pl.pallas_call(_denseK_body, out_shape=_DENSE_OUT)(
                scale, h, agg, mlp_W1[i], mlp_b1[i].reshape(1, _H),
                bn_in_gamma[i].reshape(1, _H), bn_in_beta[i].reshape(1, _H),
                mlp_W2[i], mlp_b2[i].reshape(1, _H),
                outer_gamma[i - 1].reshape(1, _H),
                outer_beta[i - 1].reshape(1, _H),
                pred_W[i + 1], pred_b[i + 1].reshape(1, _OUT), score)
    return score
